# baseline (speedup 1.0000x reference)
import functools

import jax
import jax.numpy as jnp
from jax import lax
from jax.experimental import pallas as pl
from jax.experimental.pallas import tpu as pltpu
from jax.experimental.pallas import tpu_sc as plsc

_info = plsc.get_sparse_core_info()
_NC, _NS = _info.num_cores, _info.num_subcores
_NW = _NC * _NS

_B = 16384
_S = 200
_D = 64
_TS = _S // 8
_BC = _B // 128
_UNITS = _TS * _BC
_UPW = _UNITS // _NW


def _transpose_half(rows_ref, tr_v):
    lane = lax.iota(jnp.int32, 16)
    bases = [(16 * k + lane) * 128 for k in range(4)]

    @plsc.parallel_loop(0, 512, step=1, unroll=8)
    def _(b):
        off = (b >> 7) * 8192 + (b & 127)
        for k in range(4):
            v = rows_ref[b, pl.ds(16 * k, 16)]
            plsc.store_scatter(tr_v, [bases[k] + off], v)


def _make_lookup():
    mesh = plsc.VectorSubcoreMesh(core_axis_name="c", subcore_axis_name="s")

    @functools.partial(
        pl.kernel,
        mesh=mesh,
        out_type=jax.ShapeDtypeStruct((_S, 8, _BC, 1024), jnp.float32),
        scratch_types=[
            pltpu.VMEM((2, 1024), jnp.int32),
            pltpu.VMEM((2, 512, _D), jnp.float32),
            pltpu.VMEM((4 * _D * 128,), jnp.float32),
            pltpu.SemaphoreType.DMA((2,)),
            pltpu.SemaphoreType.DMA((2,)),
            pltpu.SemaphoreType.DMA,
        ],
        compiler_params=pltpu.CompilerParams(use_tc_tiling_on_sc=False,
                                             needs_layout_passes=False),
    )
    def lookup(table_hbm, x4_hbm, o5_hbm, idx_v, rows_v, tr_v, s_idx, s_gat, s_out):
        wid = lax.axis_index("s") * _NC + lax.axis_index("c")
        g0 = wid * _UPW

        def unit_tc(u):
            g = g0 + u
            return g // _BC, g % _BC

        def idx_load(u, pu):
            t, c = unit_tc(u)
            pltpu.async_copy(x4_hbm.at[t, c], idx_v.at[pu], s_idx.at[pu])

        def idx_wait(pu):
            pltpu.make_async_copy(x4_hbm.at[0, 0], idx_v.at[pu],
                                  s_idx.at[pu]).wait()

        def gather(h, pu):
            pltpu.async_copy(
                table_hbm.at[idx_v.at[pu, pl.ds(512 * h, 512)]],
                rows_v.at[h], s_gat.at[h])

        def gather_wait(h):
            pltpu.make_async_copy(
                table_hbm.at[idx_v.at[0, pl.ds(0, 512)]], rows_v.at[h],
                s_gat.at[h]).wait()

        def writes(u, h):
            t, c = unit_tc(u)
            for rr in range(4):
                s = 8 * t + 4 * h + rr
                for jo in range(8):
                    pltpu.async_copy(
                        tr_v.at[pl.ds(rr * 8192 + jo * 1024, 1024)],
                        o5_hbm.at[s, jo, c], s_out)

        def writes_drain():
            for _ in range(32):
                pltpu.make_async_copy(tr_v.at[pl.ds(0, 1024)],
                                      o5_hbm.at[0, 0, 0], s_out).wait()

        idx_load(0, 0)
        idx_wait(0)
        gather(0, 0)
        idx_load(1, 1)

        def half_step(u, pu, h, first):
            gather_wait(h)
            if h == 0:
                gather(1, pu)
            else:
                @pl.when(u < _UPW - 1)
                def _():
                    idx_wait(1 - pu)
                    gather(0, 1 - pu)

                @pl.when(u < _UPW - 2)
                def _():
                    idx_load(u + 2, pu)

            if first:
                @pl.when(u > 0)
                def _():
                    writes_drain()
            else:
                writes_drain()
            _transpose_half(rows_v.at[h], tr_v)
            writes(u, h)

        def mega(m, carry):
            u0 = 2 * m
            half_step(u0, 0, 0, True)
            half_step(u0, 0, 1, False)
            half_step(u0 + 1, 1, 0, False)
            half_step(u0 + 1, 1, 1, False)
            return carry

        lax.fori_loop(0, _UPW // 2, mega, 0)
        writes_drain()

    return lookup


def kernel(x, table):
    x4 = jnp.transpose(x.reshape(128, 128, _TS, 8), (2, 0, 3, 1))
    x4 = x4.reshape(_TS, 128, 1024).astype(jnp.int32)
    o5 = _make_lookup()(table, x4).reshape(_S, 8, _BC, 8, 128)
    out = jnp.transpose(o5, (2, 4, 0, 1, 3))
    return out.reshape(_B, _S, _D)


# speedup vs baseline: 1.0653x; 1.0653x over previous
import functools

import jax
import jax.numpy as jnp
from jax import lax
from jax.experimental import pallas as pl
from jax.experimental.pallas import tpu as pltpu
from jax.experimental.pallas import tpu_sc as plsc

_info = plsc.get_sparse_core_info()
_NC, _NS = _info.num_cores, _info.num_subcores
_NW = _NC * _NS

_B = 16384
_S = 200
_D = 64
_TS = _S // 8
_BC = _B // 128
_UNITS = _TS * _BC
_UPW = _UNITS // _NW
_PPW = _UPW // 2


def _transpose_chunk(rows_ref, tr_ref):
    lane = lax.iota(jnp.int32, 16)
    bases = []
    for k in range(4):
        j = 16 * k + lane
        bases.append((j // 8) * 2048 + (j % 8) * 128)

    @plsc.parallel_loop(0, 256, step=1, unroll=8)
    def _(b):
        off = (b >> 7) * 1024 + (b & 127)
        for k in range(4):
            v = rows_ref[b, pl.ds(16 * k, 16)]
            plsc.store_scatter(tr_ref, [bases[k] + off], v)


def _make_lookup():
    mesh = plsc.VectorSubcoreMesh(core_axis_name="c", subcore_axis_name="s")

    @functools.partial(
        pl.kernel,
        mesh=mesh,
        out_type=jax.ShapeDtypeStruct((_S, 8, _BC // 2, 2048), jnp.float32),
        scratch_types=[
            pltpu.VMEM((2, 2, 1024), jnp.int32),
            pltpu.VMEM((2, 256, _D), jnp.float32),
            pltpu.VMEM((2, 16384), jnp.float32),
            pltpu.SemaphoreType.DMA((2,)),
            pltpu.SemaphoreType.DMA((2,)),
            pltpu.SemaphoreType.DMA((2,)),
        ],
        compiler_params=pltpu.CompilerParams(use_tc_tiling_on_sc=False,
                                             needs_layout_passes=False),
    )
    def lookup(table_hbm, x4_hbm, o5_hbm, idx_v, rows_v, tr_v, s_idx, s_gat, s_out):
        wid = lax.axis_index("s") * _NC + lax.axis_index("c")
        g0 = wid * _UPW

        def pair_tc(p):
            gu = g0 + 2 * p
            return gu // _BC, gu % _BC

        def idx_load(p, pp):
            t, c = pair_tc(p)
            pltpu.async_copy(x4_hbm.at[t, pl.ds(c, 2)], idx_v.at[pp],
                             s_idx.at[pp])

        def idx_wait(pp):
            pltpu.make_async_copy(x4_hbm.at[0, pl.ds(0, 2)], idx_v.at[pp],
                                  s_idx.at[pp]).wait()

        def gathers(pp, r, pr):
            for cc in range(2):
                pltpu.async_copy(
                    table_hbm.at[idx_v.at[pp, cc, pl.ds(128 * r, 128)]],
                    rows_v.at[pr, pl.ds(128 * cc, 128)], s_gat.at[pr])

        def gathers_wait(pr):
            for cc in range(2):
                pltpu.make_async_copy(
                    table_hbm.at[idx_v.at[0, 0, pl.ds(0, 128)]],
                    rows_v.at[pr, pl.ds(0, 128)], s_gat.at[pr]).wait()

        def writes(p, r, pr):
            t, c = pair_tc(p)
            s = 8 * t + r
            cp = c // 2
            for jo in range(8):
                pltpu.async_copy(tr_v.at[pr, pl.ds(2048 * jo, 2048)],
                                 o5_hbm.at[s, jo, cp], s_out.at[pr])

        def writes_drain(pr):
            for _ in range(8):
                pltpu.make_async_copy(tr_v.at[0, pl.ds(0, 2048)],
                                      o5_hbm.at[0, 0, 0], s_out.at[pr]).wait()

        idx_load(0, 0)
        idx_wait(0)
        gathers(0, 0, 0)
        idx_load(1, 1)

        def step(p, pp, r):
            pr = r % 2
            gathers_wait(pr)
            if r < 7:
                gathers(pp, r + 1, 1 - pr)
            else:
                @pl.when(p < _PPW - 1)
                def _():
                    idx_wait(1 - pp)
                    gathers(1 - pp, 0, 1 - pr)

                @pl.when(p < _PPW - 2)
                def _():
                    idx_load(p + 2, pp)

            if r >= 2:
                writes_drain(pr)
            else:
                @pl.when(p > 0)
                def _():
                    writes_drain(pr)

            _transpose_chunk(rows_v.at[pr], tr_v.at[pr])
            writes(p, r, pr)

        def mega(m, carry):
            for pp in range(2):
                p = 2 * m + pp
                for r in range(8):
                    step(p, pp, r)
            return carry

        lax.fori_loop(0, _PPW // 2, mega, 0)
        writes_drain(0)
        writes_drain(1)

    return lookup


def kernel(x, table):
    x4 = jnp.transpose(x.reshape(128, 128, _TS, 8), (2, 0, 3, 1))
    x4 = x4.reshape(_TS, 128, 1024).astype(jnp.int32)
    o5 = _make_lookup()(table, x4).reshape(_S, 8, _BC // 2, 2, 8, 128)
    out = jnp.transpose(o5, (2, 3, 5, 0, 1, 4))
    return out.reshape(_B, _S, _D)


# speedup vs baseline: 1.1695x; 1.0978x over previous
import functools

import jax
import jax.numpy as jnp
from jax import lax
from jax.experimental import pallas as pl
from jax.experimental.pallas import tpu as pltpu
from jax.experimental.pallas import tpu_sc as plsc

_info = plsc.get_sparse_core_info()
_NC, _NS = _info.num_cores, _info.num_subcores
_NW = _NC * _NS

_CHUNK = 800
_NBUF = 2


def _make_lookup(total, dim):
    assert total % _NW == 0
    per_w = total // _NW
    assert per_w % _CHUNK == 0
    n_chunks = per_w // _CHUNK
    assert n_chunks > _NBUF

    mesh = plsc.VectorSubcoreMesh(core_axis_name="c", subcore_axis_name="s")

    @functools.partial(
        pl.kernel,
        mesh=mesh,
        out_type=jax.ShapeDtypeStruct((total, dim), jnp.float32),
        scratch_types=[
            pltpu.VMEM((_NBUF, _CHUNK), jnp.int32),
            pltpu.VMEM((_NBUF, _CHUNK, dim), jnp.float32),
            pltpu.SemaphoreType.DMA((_NBUF,)),
            pltpu.SemaphoreType.DMA((_NBUF,)),
            pltpu.SemaphoreType.DMA((_NBUF,)),
        ],
        compiler_params=pltpu.CompilerParams(use_tc_tiling_on_sc=False),
    )
    def lookup(table_hbm, idx_hbm, out_hbm, idx_v, rows_v, s_idx, s_gat, s_out):
        wid = lax.axis_index("s") * _NC + lax.axis_index("c")
        base = wid * per_w

        def idx_load(c, b):
            pltpu.async_copy(
                idx_hbm.at[pl.ds(base + c * _CHUNK, _CHUNK)], idx_v.at[b],
                s_idx.at[b])

        def idx_wait(b):
            pltpu.make_async_copy(
                idx_hbm.at[pl.ds(0, _CHUNK)], idx_v.at[b], s_idx.at[b]).wait()

        def gather(b):
            pltpu.async_copy(table_hbm.at[idx_v.at[b]], rows_v.at[b],
                             s_gat.at[b])

        def gather_wait(b):
            pltpu.make_async_copy(
                table_hbm.at[idx_v.at[b]], rows_v.at[b], s_gat.at[b]).wait()

        def write(c, b):
            pltpu.async_copy(
                rows_v.at[b], out_hbm.at[pl.ds(base + c * _CHUNK, _CHUNK)],
                s_out.at[b])

        def write_wait(b):
            pltpu.make_async_copy(
                rows_v.at[b], out_hbm.at[pl.ds(0, _CHUNK)], s_out.at[b]).wait()

        for b in range(_NBUF):
            idx_load(b, b)

        def step(g, carry):
            for b in range(_NBUF):
                c = g * _NBUF + b
                idx_wait(b)

                @pl.when(g > 0)
                def _():
                    write_wait(b)

                gather(b)
                gather_wait(b)
                write(c, b)

                @pl.when(c + _NBUF < n_chunks)
                def _():
                    idx_load(c + _NBUF, b)

            return carry

        lax.fori_loop(0, n_chunks // _NBUF, step, 0)

        for b in range(_NBUF):
            write_wait(b)

    return lookup


def kernel(x, table):
    batch, seq = x.shape
    total = batch * seq
    dim = table.shape[1]
    x_flat = x.reshape(total).astype(jnp.int32)
    out = _make_lookup(total, dim)(table, x_flat)
    return out.reshape(batch, seq, dim)
